# Initial kernel scaffold; baseline (speedup 1.0000x reference)
import jax
import jax.numpy as jnp
from jax.experimental import pallas as pl


def kernel(node_attr, edge_attr, global_attr, edge_index, ng_index, eg_index, W1, b1, W2, b2):
    raise NotImplementedError("write your pallas kernel here")


# speedup vs baseline: 6.3508x; 6.3508x over previous
import functools

import jax
import jax.numpy as jnp
from jax import lax
from jax.experimental import pallas as pl
from jax.experimental.pallas import tpu as pltpu
from jax.experimental.pallas import tpu_sc as plsc

_N_NODES = 10000
_N_EDGES = 320000
_D_FEAT = 128
_D_EDGE = 16
_D_GLOBAL = 16
_N_BATCHES = 8
_IN_DIM = _D_FEAT + _D_EDGE + _D_GLOBAL
_LATENT = 32
_OUT_DIM = 128

_NC = 2
_NS = 16
_NW = _NC * _NS
_EDGES_PER_W = _N_EDGES // _NW
_CHUNK = 80
_K = 5
_ROUNDS = _EDGES_PER_W // (_CHUNK * _K)
_ROWS_PER_TILE = 624
_ROWS_TAIL = _N_NODES - _ROWS_PER_TILE * _NS

_BLK = 2000
_GRID = _N_NODES // _BLK


def _sc_scatter_mean_partials(recv, edge_attr):
    mesh = plsc.VectorSubcoreMesh(core_axis_name="c", subcore_axis_name="s")

    @functools.partial(
        pl.kernel,
        out_type=(
            jax.ShapeDtypeStruct((_NC, _N_NODES, _D_EDGE), jnp.float32),
            jax.ShapeDtypeStruct((_NC, _N_NODES, _D_EDGE), jnp.float32),
        ),
        mesh=mesh,
        scratch_types=[
            pltpu.VMEM_SHARED((_N_NODES, _D_EDGE), jnp.float32),
            pltpu.VMEM_SHARED((_N_NODES, _D_EDGE), jnp.float32),
            pltpu.VMEM((_K, _CHUNK), jnp.int32),
            pltpu.VMEM((_K, _CHUNK, _D_EDGE), jnp.float32),
            pltpu.VMEM((_CHUNK, _D_EDGE), jnp.float32),
            pltpu.VMEM((_ROWS_PER_TILE, _D_EDGE), jnp.float32),
            pltpu.SemaphoreType.DMA,
            pltpu.SemaphoreType.DMA,
        ],
        compiler_params=pltpu.CompilerParams(use_tc_tiling_on_sc=False),
    )
    def scatter_kernel(recv_hbm, edge_hbm, sums_hbm, cnts_hbm,
                       sums_sp, cnts_sp, idx_v, rows_v, ones_v, zero_v,
                       sem_g, sem_s):
        cid = lax.axis_index("c")
        sid = lax.axis_index("s")
        wid = sid * _NC + cid
        base = wid * _EDGES_PER_W

        zvec = jnp.zeros((16,), jnp.float32)
        ovec = jnp.ones((16,), jnp.float32)

        @pl.loop(0, _ROWS_PER_TILE)
        def _zero_init(i):
            zero_v[i] = zvec

        @pl.loop(0, _CHUNK)
        def _ones_init(i):
            ones_v[i] = ovec

        row0 = sid * _ROWS_PER_TILE
        pltpu.sync_copy(zero_v, sums_sp.at[pl.ds(row0, _ROWS_PER_TILE), :])
        pltpu.sync_copy(zero_v, cnts_sp.at[pl.ds(row0, _ROWS_PER_TILE), :])

        tail0 = _ROWS_PER_TILE * _NS

        @pl.when(sid == _NS - 1)
        def _zero_tail():
            pltpu.sync_copy(zero_v.at[pl.ds(0, _ROWS_TAIL), :],
                            sums_sp.at[pl.ds(tail0, _ROWS_TAIL), :])
            pltpu.sync_copy(zero_v.at[pl.ds(0, _ROWS_TAIL), :],
                            cnts_sp.at[pl.ds(tail0, _ROWS_TAIL), :])

        plsc.subcore_barrier()

        @pl.loop(0, _ROUNDS)
        def _round(r):
            c0 = base + r * (_CHUNK * _K)
            gathers = []
            for b in range(_K):
                off = c0 + b * _CHUNK
                gathers.append(pltpu.async_copy(
                    recv_hbm.at[pl.ds(off, _CHUNK)], idx_v.at[b], sem_g))
                gathers.append(pltpu.async_copy(
                    edge_hbm.at[pl.ds(off, _CHUNK), :], rows_v.at[b], sem_g))
            for d in gathers:
                d.wait()
            scatters = []
            for b in range(_K):
                scatters.append(pltpu.async_copy(
                    rows_v.at[b], sums_sp.at[idx_v.at[b]], sem_s, add=True))
                scatters.append(pltpu.async_copy(
                    ones_v, cnts_sp.at[idx_v.at[b]], sem_s, add=True))
            for d in scatters:
                d.wait()

        plsc.subcore_barrier()
        pltpu.sync_copy(sums_sp.at[pl.ds(row0, _ROWS_PER_TILE), :],
                        sums_hbm.at[cid, pl.ds(row0, _ROWS_PER_TILE), :])
        pltpu.sync_copy(cnts_sp.at[pl.ds(row0, _ROWS_PER_TILE), :],
                        cnts_hbm.at[cid, pl.ds(row0, _ROWS_PER_TILE), :])

        @pl.when(sid == _NS - 1)
        def _write_tail():
            pltpu.sync_copy(sums_sp.at[pl.ds(tail0, _ROWS_TAIL), :],
                            sums_hbm.at[cid, pl.ds(tail0, _ROWS_TAIL), :])
            pltpu.sync_copy(cnts_sp.at[pl.ds(tail0, _ROWS_TAIL), :],
                            cnts_hbm.at[cid, pl.ds(tail0, _ROWS_TAIL), :])

    return scatter_kernel(recv, edge_attr)


def _tc_mlp_kernel(na_ref, s_ref, c_ref, g_ref, ng_ref,
                   W1_ref, b1_ref, W2_ref, b2_ref, out_ref):
    s = s_ref[0] + s_ref[1]
    c = c_ref[0] + c_ref[1]
    agg = s / jnp.maximum(c, 1.0)
    na = na_ref[...]
    ng = ng_ref[0, 0]
    iota = lax.broadcasted_iota(jnp.int32, (_BLK, _N_BATCHES), 1)
    onehot = (ng[:, None] == iota).astype(jnp.float32)
    gW = jnp.dot(g_ref[...], W1_ref[_D_FEAT + _D_EDGE:, :],
                 preferred_element_type=jnp.float32)
    h = (jnp.dot(na, W1_ref[:_D_FEAT, :], preferred_element_type=jnp.float32)
         + jnp.dot(agg, W1_ref[_D_FEAT:_D_FEAT + _D_EDGE, :],
                   preferred_element_type=jnp.float32)
         + jnp.dot(onehot, gW, preferred_element_type=jnp.float32)
         + b1_ref[...])
    h = jnp.maximum(h, 0.0)
    out_ref[...] = jnp.dot(h, W2_ref[...],
                           preferred_element_type=jnp.float32) + b2_ref[...]


def _tc_mlp(node_attr, sums, cnts, global_attr, ng3, W1, b1r, W2, b2r):
    return pl.pallas_call(
        _tc_mlp_kernel,
        grid=(_GRID,),
        in_specs=[
            pl.BlockSpec((_BLK, _D_FEAT), lambda i: (i, 0)),
            pl.BlockSpec((_NC, _BLK, _D_EDGE), lambda i: (0, i, 0)),
            pl.BlockSpec((_NC, _BLK, _D_EDGE), lambda i: (0, i, 0)),
            pl.BlockSpec((_N_BATCHES, _D_GLOBAL), lambda i: (0, 0)),
            pl.BlockSpec((1, 1, _BLK), lambda i: (i, 0, 0)),
            pl.BlockSpec((_IN_DIM, _LATENT), lambda i: (0, 0)),
            pl.BlockSpec((1, _LATENT), lambda i: (0, 0)),
            pl.BlockSpec((_LATENT, _OUT_DIM), lambda i: (0, 0)),
            pl.BlockSpec((1, _OUT_DIM), lambda i: (0, 0)),
        ],
        out_specs=pl.BlockSpec((_BLK, _OUT_DIM), lambda i: (i, 0)),
        out_shape=jax.ShapeDtypeStruct((_N_NODES, _OUT_DIM), jnp.float32),
    )(node_attr, sums, cnts, global_attr, ng3, W1, b1r, W2, b2r)


def kernel(node_attr, edge_attr, global_attr, edge_index, ng_index, eg_index,
           W1, b1, W2, b2):
    recv = edge_index[1]
    sums, cnts = _sc_scatter_mean_partials(recv, edge_attr)
    ng3 = ng_index.reshape(_GRID, 1, _BLK)
    return _tc_mlp(node_attr, sums, cnts, global_attr, ng3,
                   W1, b1.reshape(1, -1), W2, b2.reshape(1, -1))


# speedup vs baseline: 6.5370x; 1.0293x over previous
import functools

import jax
import jax.numpy as jnp
from jax import lax
from jax.experimental import pallas as pl
from jax.experimental.pallas import tpu as pltpu
from jax.experimental.pallas import tpu_sc as plsc

_N_NODES = 10000
_N_EDGES = 320000
_D_FEAT = 128
_D_EDGE = 16
_D_GLOBAL = 16
_N_BATCHES = 8
_IN_DIM = _D_FEAT + _D_EDGE + _D_GLOBAL
_LATENT = 32
_OUT_DIM = 128

_NC = 2
_NS = 16
_NW = _NC * _NS
_CHUNK = 128
_NCHUNKS = _N_EDGES // _CHUNK
_ROWS_MAIN = _NCHUNKS // _NW
_ROWS_EXTRA = _NCHUNKS - _ROWS_MAIN * _NW
_K = 6
_ROUNDS = _ROWS_MAIN // _K
_ZROWS = 624
_ZTAIL = _N_NODES - _ZROWS * _NS

_BLK = 2000
_GRID = _N_NODES // _BLK


def _sc_scatter_mean_partials(recv2, edge_attr, zeros_c, ones_c):
    mesh = plsc.VectorSubcoreMesh(core_axis_name="c", subcore_axis_name="s")

    @functools.partial(
        pl.kernel,
        out_type=(
            jax.ShapeDtypeStruct((_NC, _N_NODES, _D_EDGE), jnp.float32),
            jax.ShapeDtypeStruct((_NC, _N_NODES, _D_EDGE), jnp.float32),
        ),
        mesh=mesh,
        scratch_types=[
            pltpu.VMEM_SHARED((_N_NODES, _D_EDGE), jnp.float32),
            pltpu.VMEM_SHARED((_N_NODES, _D_EDGE), jnp.float32),
            pltpu.VMEM((_K, _CHUNK), jnp.int32),
            pltpu.VMEM((_K, _CHUNK), jnp.int32),
            pltpu.VMEM((_K * _CHUNK, _D_EDGE), jnp.float32),
            pltpu.VMEM((_K * _CHUNK, _D_EDGE), jnp.float32),
            pltpu.VMEM((_CHUNK, _D_EDGE), jnp.float32),
            pltpu.SemaphoreType.DMA,
            pltpu.SemaphoreType.DMA,
            pltpu.SemaphoreType.DMA,
            pltpu.SemaphoreType.DMA,
        ],
        compiler_params=pltpu.CompilerParams(use_tc_tiling_on_sc=False),
    )
    def scatter_kernel(recv_hbm, edge_hbm, zeros_hbm, ones_hbm,
                       sums_hbm, cnts_hbm,
                       sums_sp, cnts_sp, idx0, idx1, rows0, rows1, ones_v,
                       sem_g0, sem_g1, sem_s0, sem_s1):
        cid = lax.axis_index("c")
        sid = lax.axis_index("s")
        wid = sid * _NC + cid
        crow0 = wid * _ROWS_MAIN

        pltpu.sync_copy(ones_hbm, ones_v)

        row0 = sid * _ZROWS
        pltpu.sync_copy(zeros_hbm, sums_sp.at[pl.ds(row0, _ZROWS), :])
        pltpu.sync_copy(zeros_hbm, cnts_sp.at[pl.ds(row0, _ZROWS), :])
        tail0 = _ZROWS * _NS

        @pl.when(sid == _NS - 1)
        def _zero_tail():
            pltpu.sync_copy(zeros_hbm.at[pl.ds(0, _ZTAIL), :],
                            sums_sp.at[pl.ds(tail0, _ZTAIL), :])
            pltpu.sync_copy(zeros_hbm.at[pl.ds(0, _ZTAIL), :],
                            cnts_sp.at[pl.ds(tail0, _ZTAIL), :])

        plsc.subcore_barrier()

        idx_b = (idx0, idx1)
        rows_b = (rows0, rows1)
        sem_g = (sem_g0, sem_g1)
        sem_s = (sem_s0, sem_s1)

        def gather_round(rr, s):
            g1 = pltpu.async_copy(recv_hbm.at[pl.ds(rr, _K), :],
                                  idx_b[s], sem_g[s])
            g2 = pltpu.async_copy(edge_hbm.at[pl.ds(rr * _CHUNK, _K * _CHUNK), :],
                                  rows_b[s], sem_g[s])
            return (g1, g2)

        def scatter_round(s):
            ds = []
            for b in range(_K):
                ds.append(pltpu.async_copy(
                    rows_b[s].at[pl.ds(b * _CHUNK, _CHUNK), :],
                    sums_sp.at[idx_b[s].at[b]], sem_s[s], add=True))
                ds.append(pltpu.async_copy(
                    ones_v, cnts_sp.at[idx_b[s].at[b]], sem_s[s], add=True))
            return ds

        @pl.loop(0, _ROUNDS - 1, step=2)
        def _pair(r):
            rr0 = crow0 + r * _K
            g0 = gather_round(rr0, 0)
            g1 = gather_round(rr0 + _K, 1)
            for d in g0:
                d.wait()
            s0 = scatter_round(0)
            for d in g1:
                d.wait()
            s1 = scatter_round(1)
            for d in s0:
                d.wait()
            for d in s1:
                d.wait()

        gl = gather_round(crow0 + (_ROUNDS - 1) * _K, 0)

        @pl.when(wid < _ROWS_EXTRA)
        def _extra_gather():
            xrow = _ROWS_MAIN * _NW + wid
            pltpu.async_copy(recv_hbm.at[pl.ds(xrow, 1), :],
                             idx1.at[pl.ds(0, 1), :], sem_g1).wait()
            pltpu.async_copy(edge_hbm.at[pl.ds(xrow * _CHUNK, _CHUNK), :],
                             rows1.at[pl.ds(0, _CHUNK), :], sem_g1).wait()
            d1 = pltpu.async_copy(rows1.at[pl.ds(0, _CHUNK), :],
                                  sums_sp.at[idx1.at[0]], sem_s1, add=True)
            d2 = pltpu.async_copy(ones_v, cnts_sp.at[idx1.at[0]],
                                  sem_s1, add=True)
            d1.wait()
            d2.wait()

        for d in gl:
            d.wait()
        for d in scatter_round(0):
            d.wait()

        plsc.subcore_barrier()
        pltpu.sync_copy(sums_sp.at[pl.ds(row0, _ZROWS), :],
                        sums_hbm.at[cid, pl.ds(row0, _ZROWS), :])
        pltpu.sync_copy(cnts_sp.at[pl.ds(row0, _ZROWS), :],
                        cnts_hbm.at[cid, pl.ds(row0, _ZROWS), :])

        @pl.when(sid == _NS - 1)
        def _write_tail():
            pltpu.sync_copy(sums_sp.at[pl.ds(tail0, _ZTAIL), :],
                            sums_hbm.at[cid, pl.ds(tail0, _ZTAIL), :])
            pltpu.sync_copy(cnts_sp.at[pl.ds(tail0, _ZTAIL), :],
                            cnts_hbm.at[cid, pl.ds(tail0, _ZTAIL), :])

    return scatter_kernel(recv2, edge_attr, zeros_c, ones_c)


def _tc_mlp_kernel(na_ref, s_ref, c_ref, g_ref, ng_ref,
                   W1_ref, b1_ref, W2_ref, b2_ref, out_ref):
    s = s_ref[0] + s_ref[1]
    c = c_ref[0] + c_ref[1]
    agg = s / jnp.maximum(c, 1.0)
    na = na_ref[...]
    ng = ng_ref[0, 0]
    iota = lax.broadcasted_iota(jnp.int32, (_BLK, _N_BATCHES), 1)
    onehot = (ng[:, None] == iota).astype(jnp.float32)
    gW = jnp.dot(g_ref[...], W1_ref[_D_FEAT + _D_EDGE:, :],
                 preferred_element_type=jnp.float32)
    h = (jnp.dot(na, W1_ref[:_D_FEAT, :], preferred_element_type=jnp.float32)
         + jnp.dot(agg, W1_ref[_D_FEAT:_D_FEAT + _D_EDGE, :],
                   preferred_element_type=jnp.float32)
         + jnp.dot(onehot, gW, preferred_element_type=jnp.float32)
         + b1_ref[...])
    h = jnp.maximum(h, 0.0)
    out_ref[...] = jnp.dot(h, W2_ref[...],
                           preferred_element_type=jnp.float32) + b2_ref[...]


def _tc_mlp(node_attr, sums, cnts, global_attr, ng3, W1, b1r, W2, b2r):
    return pl.pallas_call(
        _tc_mlp_kernel,
        grid=(_GRID,),
        in_specs=[
            pl.BlockSpec((_BLK, _D_FEAT), lambda i: (i, 0)),
            pl.BlockSpec((_NC, _BLK, _D_EDGE), lambda i: (0, i, 0)),
            pl.BlockSpec((_NC, _BLK, _D_EDGE), lambda i: (0, i, 0)),
            pl.BlockSpec((_N_BATCHES, _D_GLOBAL), lambda i: (0, 0)),
            pl.BlockSpec((1, 1, _BLK), lambda i: (i, 0, 0)),
            pl.BlockSpec((_IN_DIM, _LATENT), lambda i: (0, 0)),
            pl.BlockSpec((1, _LATENT), lambda i: (0, 0)),
            pl.BlockSpec((_LATENT, _OUT_DIM), lambda i: (0, 0)),
            pl.BlockSpec((1, _OUT_DIM), lambda i: (0, 0)),
        ],
        out_specs=pl.BlockSpec((_BLK, _OUT_DIM), lambda i: (i, 0)),
        out_shape=jax.ShapeDtypeStruct((_N_NODES, _OUT_DIM), jnp.float32),
    )(node_attr, sums, cnts, global_attr, ng3, W1, b1r, W2, b2r)


def kernel(node_attr, edge_attr, global_attr, edge_index, ng_index, eg_index,
           W1, b1, W2, b2):
    recv2 = edge_index[1].reshape(_NCHUNKS, _CHUNK)
    zeros_c = jnp.zeros((_ZROWS, _D_EDGE), jnp.float32)
    ones_c = jnp.ones((_CHUNK, _D_EDGE), jnp.float32)
    sums, cnts = _sc_scatter_mean_partials(recv2, edge_attr, zeros_c, ones_c)
    ng3 = ng_index.reshape(_GRID, 1, _BLK)
    return _tc_mlp(node_attr, sums, cnts, global_attr, ng3,
                   W1, b1.reshape(1, -1), W2, b2.reshape(1, -1))


# speedup vs baseline: 6.6834x; 1.0224x over previous
import functools

import jax
import jax.numpy as jnp
from jax import lax
from jax.experimental import pallas as pl
from jax.experimental.pallas import tpu as pltpu
from jax.experimental.pallas import tpu_sc as plsc

_N_NODES = 10000
_N_EDGES = 320000
_D_FEAT = 128
_D_EDGE = 16
_D_CNT = 8
_D_GLOBAL = 16
_N_BATCHES = 8
_IN_DIM = _D_FEAT + _D_EDGE + _D_GLOBAL
_LATENT = 32
_OUT_DIM = 128

_NC = 2
_NS = 16
_NW = _NC * _NS
_EPW = _N_EDGES // _NW
_CHUNK = 128
_NCH = _EPW // _CHUNK
_TAIL = _EPW - _NCH * _CHUNK
_K = 6
_ROUNDS = _NCH // _K
_ZROWS = 624
_ZTAIL = _N_NODES - _ZROWS * _NS

_BLK = 2000
_GRID = _N_NODES // _BLK


def _sc_scatter_mean_partials(edge_index, edge_attr, zeros16, zeros8, ones8):
    mesh = plsc.VectorSubcoreMesh(core_axis_name="c", subcore_axis_name="s")

    @functools.partial(
        pl.kernel,
        out_type=(
            jax.ShapeDtypeStruct((_NC, _N_NODES, _D_EDGE), jnp.float32),
            jax.ShapeDtypeStruct((_NC, _N_NODES, _D_CNT), jnp.float32),
        ),
        mesh=mesh,
        scratch_types=[
            pltpu.VMEM_SHARED((_N_NODES, _D_EDGE), jnp.float32),
            pltpu.VMEM_SHARED((_N_NODES, _D_CNT), jnp.float32),
            pltpu.VMEM((_K, _CHUNK), jnp.int32),
            pltpu.VMEM((_K, _CHUNK), jnp.int32),
            pltpu.VMEM((_K * _CHUNK, _D_EDGE), jnp.float32),
            pltpu.VMEM((_K * _CHUNK, _D_EDGE), jnp.float32),
            pltpu.VMEM((1, _TAIL), jnp.int32),
            pltpu.VMEM((_CHUNK, _D_CNT), jnp.float32),
            pltpu.SemaphoreType.DMA,
            pltpu.SemaphoreType.DMA,
            pltpu.SemaphoreType.DMA,
            pltpu.SemaphoreType.DMA,
        ],
        compiler_params=pltpu.CompilerParams(use_tc_tiling_on_sc=False),
    )
    def scatter_kernel(eidx_hbm, edge_hbm, z16_hbm, z8_hbm, ones_hbm,
                       sums_hbm, cnts_hbm,
                       sums_sp, cnts_sp, idx0, idx1, rows0, rows1, tidx, ones_v,
                       sem_g0, sem_g1, sem_s0, sem_s1):
        cid = lax.axis_index("c")
        sid = lax.axis_index("s")
        wid = sid * _NC + cid
        woff = wid * _EPW

        pltpu.sync_copy(ones_hbm, ones_v)

        row0 = sid * _ZROWS
        pltpu.sync_copy(z16_hbm, sums_sp.at[pl.ds(row0, _ZROWS), :])
        pltpu.sync_copy(z8_hbm, cnts_sp.at[pl.ds(row0, _ZROWS), :])
        tail0 = _ZROWS * _NS

        @pl.when(sid == _NS - 1)
        def _zero_tail():
            pltpu.sync_copy(z16_hbm.at[pl.ds(0, _ZTAIL), :],
                            sums_sp.at[pl.ds(tail0, _ZTAIL), :])
            pltpu.sync_copy(z8_hbm.at[pl.ds(0, _ZTAIL), :],
                            cnts_sp.at[pl.ds(tail0, _ZTAIL), :])

        plsc.subcore_barrier()

        idx_b = (idx0, idx1)
        rows_b = (rows0, rows1)
        sem_g = (sem_g0, sem_g1)
        sem_s = (sem_s0, sem_s1)

        def gather_round(r, s):
            off = woff + r * (_K * _CHUNK)
            ds_ = [pltpu.async_copy(edge_hbm.at[pl.ds(off, _K * _CHUNK), :],
                                    rows_b[s], sem_g[s])]
            for b in range(_K):
                ds_.append(pltpu.async_copy(
                    eidx_hbm.at[1, pl.ds(off + b * _CHUNK, _CHUNK)],
                    idx_b[s].at[b], sem_g[s]))
            return ds_

        def scatter_round(s):
            ds_ = []
            for b in range(_K):
                ds_.append(pltpu.async_copy(
                    rows_b[s].at[pl.ds(b * _CHUNK, _CHUNK), :],
                    sums_sp.at[idx_b[s].at[b]], sem_s[s], add=True))
                ds_.append(pltpu.async_copy(
                    ones_v, cnts_sp.at[idx_b[s].at[b]], sem_s[s], add=True))
            return ds_

        @pl.loop(0, _ROUNDS - 1, step=2)
        def _pair(r):
            g0 = gather_round(r, 0)
            g1 = gather_round(r + 1, 1)
            for d in g0:
                d.wait()
            s0 = scatter_round(0)
            for d in g1:
                d.wait()
            s1 = scatter_round(1)
            for d in s0:
                d.wait()
            for d in s1:
                d.wait()

        gl = gather_round(_ROUNDS - 1, 0)
        pltpu.async_copy(eidx_hbm.at[1, pl.ds(woff + _NCH * _CHUNK, _TAIL)],
                         tidx.at[0], sem_g1).wait()
        pltpu.async_copy(edge_hbm.at[pl.ds(woff + _NCH * _CHUNK, _TAIL), :],
                         rows1.at[pl.ds(0, _TAIL), :], sem_g1).wait()
        dt1 = pltpu.async_copy(rows1.at[pl.ds(0, _TAIL), :],
                               sums_sp.at[tidx.at[0]], sem_s1, add=True)
        dt2 = pltpu.async_copy(ones_v.at[pl.ds(0, _TAIL), :],
                               cnts_sp.at[tidx.at[0]], sem_s1, add=True)
        for d in gl:
            d.wait()
        sl = scatter_round(0)
        dt1.wait()
        dt2.wait()
        for d in sl:
            d.wait()

        plsc.subcore_barrier()
        pltpu.sync_copy(sums_sp.at[pl.ds(row0, _ZROWS), :],
                        sums_hbm.at[cid, pl.ds(row0, _ZROWS), :])
        pltpu.sync_copy(cnts_sp.at[pl.ds(row0, _ZROWS), :],
                        cnts_hbm.at[cid, pl.ds(row0, _ZROWS), :])

        @pl.when(sid == _NS - 1)
        def _write_tail():
            pltpu.sync_copy(sums_sp.at[pl.ds(tail0, _ZTAIL), :],
                            sums_hbm.at[cid, pl.ds(tail0, _ZTAIL), :])
            pltpu.sync_copy(cnts_sp.at[pl.ds(tail0, _ZTAIL), :],
                            cnts_hbm.at[cid, pl.ds(tail0, _ZTAIL), :])

    return scatter_kernel(edge_index, edge_attr, zeros16, zeros8, ones8)


def _tc_mlp_kernel(na_ref, s_ref, c_ref, g_ref, ng_ref,
                   W1_ref, b1_ref, W2_ref, b2_ref, out_ref):
    s = s_ref[0] + s_ref[1]
    c = c_ref[0] + c_ref[1]
    agg = s / jnp.maximum(c[:, 0:1], 1.0)
    na = na_ref[...]
    ng = ng_ref[0, 0]
    iota = lax.broadcasted_iota(jnp.int32, (_BLK, _N_BATCHES), 1)
    onehot = (ng[:, None] == iota).astype(jnp.float32)
    gW = jnp.dot(g_ref[...], W1_ref[_D_FEAT + _D_EDGE:, :],
                 preferred_element_type=jnp.float32)
    h = (jnp.dot(na, W1_ref[:_D_FEAT, :], preferred_element_type=jnp.float32)
         + jnp.dot(agg, W1_ref[_D_FEAT:_D_FEAT + _D_EDGE, :],
                   preferred_element_type=jnp.float32)
         + jnp.dot(onehot, gW, preferred_element_type=jnp.float32)
         + b1_ref[...])
    h = jnp.maximum(h, 0.0)
    out_ref[...] = jnp.dot(h, W2_ref[...],
                           preferred_element_type=jnp.float32) + b2_ref[...]


def _tc_mlp(node_attr, sums, cnts, global_attr, ng3, W1, b1r, W2, b2r):
    return pl.pallas_call(
        _tc_mlp_kernel,
        grid=(_GRID,),
        in_specs=[
            pl.BlockSpec((_BLK, _D_FEAT), lambda i: (i, 0)),
            pl.BlockSpec((_NC, _BLK, _D_EDGE), lambda i: (0, i, 0)),
            pl.BlockSpec((_NC, _BLK, _D_CNT), lambda i: (0, i, 0)),
            pl.BlockSpec((_N_BATCHES, _D_GLOBAL), lambda i: (0, 0)),
            pl.BlockSpec((1, 1, _BLK), lambda i: (i, 0, 0)),
            pl.BlockSpec((_IN_DIM, _LATENT), lambda i: (0, 0)),
            pl.BlockSpec((1, _LATENT), lambda i: (0, 0)),
            pl.BlockSpec((_LATENT, _OUT_DIM), lambda i: (0, 0)),
            pl.BlockSpec((1, _OUT_DIM), lambda i: (0, 0)),
        ],
        out_specs=pl.BlockSpec((_BLK, _OUT_DIM), lambda i: (i, 0)),
        out_shape=jax.ShapeDtypeStruct((_N_NODES, _OUT_DIM), jnp.float32),
    )(node_attr, sums, cnts, global_attr, ng3, W1, b1r, W2, b2r)


def kernel(node_attr, edge_attr, global_attr, edge_index, ng_index, eg_index,
           W1, b1, W2, b2):
    zeros16 = jnp.zeros((_ZROWS, _D_EDGE), jnp.float32)
    zeros8 = jnp.zeros((_ZROWS, _D_CNT), jnp.float32)
    ones8 = jnp.ones((_CHUNK, _D_CNT), jnp.float32)
    sums, cnts = _sc_scatter_mean_partials(edge_index, edge_attr,
                                           zeros16, zeros8, ones8)
    ng3 = ng_index.reshape(_GRID, 1, _BLK)
    return _tc_mlp(node_attr, sums, cnts, global_attr, ng3,
                   W1, b1.reshape(1, -1), W2, b2.reshape(1, -1))


# speedup vs baseline: 7.4953x; 1.1215x over previous
import functools

import jax
import jax.numpy as jnp
from jax import lax
from jax.experimental import pallas as pl
from jax.experimental.pallas import tpu as pltpu
from jax.experimental.pallas import tpu_sc as plsc

_N_NODES = 10000
_N_EDGES = 320000
_D_FEAT = 128
_D_EDGE = 16
_D_GLOBAL = 16
_N_BATCHES = 8
_IN_DIM = _D_FEAT + _D_EDGE + _D_GLOBAL
_LATENT = 32
_OUT_DIM = 128

_NC = 2
_NS = 16
_NCH = _N_EDGES // 128
_HROWS = _NCH // _NC
_RROWS = 125
_NROUND = _HROWS // _RROWS
_CROWS = _HROWS // _NS
_CEXTRA = _HROWS - _CROWS * _NS

_BLK = 2048
_GRID = -(-_N_NODES // _BLK)


def _sc_scatter_mean_partials(e4, ei3, zeros_n):
    mesh = plsc.VectorSubcoreMesh(core_axis_name="c", subcore_axis_name="s")

    @functools.partial(
        pl.kernel,
        out_type=(
            jax.ShapeDtypeStruct((_NC, _NS, _N_NODES), jnp.float32),
            jax.ShapeDtypeStruct((_NC, _NS, _N_NODES), jnp.float32),
        ),
        mesh=mesh,
        scratch_types=[
            pltpu.VMEM((_N_NODES,), jnp.float32),
            pltpu.VMEM((_N_NODES,), jnp.float32),
            pltpu.VMEM((_RROWS, 128), jnp.float32),
            pltpu.VMEM((_RROWS, 128), jnp.float32),
            pltpu.VMEM((_RROWS, 128), jnp.int32),
            pltpu.VMEM((_RROWS, 128), jnp.int32),
            pltpu.SemaphoreType.DMA,
            pltpu.SemaphoreType.DMA,
        ],
        compiler_params=pltpu.CompilerParams(use_tc_tiling_on_sc=False,
                                             needs_layout_passes=False),
    )
    def scatter_kernel(e4_hbm, ei3_hbm, zn_hbm, sums_hbm, cnts_hbm,
                       acc, cacc, strip0, strip1, recv0, recv1,
                       sem0, sem1):
        cid = lax.axis_index("c")
        sid = lax.axis_index("s")
        tr = sid // 8
        l0 = (sid % 8) * 128
        half0 = cid * _HROWS

        pltpu.sync_copy(zn_hbm, acc)
        pltpu.sync_copy(zn_hbm, cacc)

        strip_b = (strip0, strip1)
        recv_b = (recv0, recv1)
        sem = (sem0, sem1)
        ones16 = jnp.ones((16,), jnp.float32)

        def gather_round(rr, s):
            c0 = half0 + rr * _RROWS
            g1 = pltpu.async_copy(
                e4_hbm.at[tr, pl.ds(c0, _RROWS), pl.ds(l0, 128)],
                strip_b[s], sem[s])
            g2 = pltpu.async_copy(
                ei3_hbm.at[pl.ds(c0, _RROWS), pl.ds(128, 128)],
                recv_b[s], sem[s])
            return (g1, g2)

        def accum_round(s):
            sv = strip_b[s]
            rv = recv_b[s]

            @pl.loop(0, _RROWS)
            def _row(i):
                for g in range(8):
                    idxv = rv[i, pl.ds(16 * g, 16)]
                    valv = sv[i, pl.ds(16 * g, 16)]
                    plsc.addupdate_scatter(acc, [idxv], valv)

        @pl.loop(0, _NROUND, step=2)
        def _pair(rr):
            g0 = gather_round(rr, 0)
            g1 = gather_round(rr + 1, 1)
            for d in g0:
                d.wait()
            accum_round(0)
            for d in g1:
                d.wait()
            accum_round(1)

        crow0 = half0 + sid * _CROWS
        dc = pltpu.async_copy(ei3_hbm.at[pl.ds(crow0, _CROWS), pl.ds(128, 128)],
                              recv0.at[pl.ds(0, _CROWS), :], sem0)
        dc.wait()

        @pl.loop(0, _CROWS)
        def _crow(i):
            for g in range(8):
                idxv = recv0[i, pl.ds(16 * g, 16)]
                plsc.addupdate_scatter(cacc, [idxv], ones16)

        @pl.when(sid == _NS - 1)
        def _cextra():
            dx = pltpu.async_copy(
                ei3_hbm.at[pl.ds(half0 + _NS * _CROWS, _CEXTRA), pl.ds(128, 128)],
                recv1.at[pl.ds(0, _CEXTRA), :], sem1)
            dx.wait()

            @pl.loop(0, _CEXTRA)
            def _xrow(i):
                for g in range(8):
                    idxv = recv1[i, pl.ds(16 * g, 16)]
                    plsc.addupdate_scatter(cacc, [idxv], ones16)

        pltpu.sync_copy(acc, sums_hbm.at[cid, sid, :])
        pltpu.sync_copy(cacc, cnts_hbm.at[cid, sid, :])

    return scatter_kernel(e4, ei3, zeros_n)


def _tc_mlp_kernel(na_ref, s_ref, c_ref, g_ref, ng_ref,
                   W1_ref, b1_ref, W2_ref, b2_ref, out_ref):
    sT = s_ref[0] + s_ref[1]
    cnt = jnp.sum(c_ref[...], axis=(0, 1))
    aggT = sT / jnp.maximum(cnt, 1.0)[None, :]
    na = na_ref[...]
    ng = ng_ref[0]
    iota = lax.broadcasted_iota(jnp.int32, (_BLK, _N_BATCHES), 1)
    onehot = (ng[:, None] == iota).astype(jnp.float32)
    gW = jnp.dot(g_ref[...], W1_ref[_D_FEAT + _D_EDGE:, :],
                 preferred_element_type=jnp.float32)
    h_e = lax.dot_general(aggT, W1_ref[_D_FEAT:_D_FEAT + _D_EDGE, :],
                          ((( 0,), (0,)), ((), ())),
                          preferred_element_type=jnp.float32)
    h = (jnp.dot(na, W1_ref[:_D_FEAT, :], preferred_element_type=jnp.float32)
         + h_e
         + jnp.dot(onehot, gW, preferred_element_type=jnp.float32)
         + b1_ref[...])
    h = jnp.maximum(h, 0.0)
    out_ref[...] = jnp.dot(h, W2_ref[...],
                           preferred_element_type=jnp.float32) + b2_ref[...]


def _tc_mlp(node_attr, sums, cnts, global_attr, ng2, W1, b1r, W2, b2r):
    return pl.pallas_call(
        _tc_mlp_kernel,
        grid=(_GRID,),
        in_specs=[
            pl.BlockSpec((_BLK, _D_FEAT), lambda i: (i, 0)),
            pl.BlockSpec((_NC, _NS, _BLK), lambda i: (0, 0, i)),
            pl.BlockSpec((_NC, _NS, _BLK), lambda i: (0, 0, i)),
            pl.BlockSpec((_N_BATCHES, _D_GLOBAL), lambda i: (0, 0)),
            pl.BlockSpec((1, _BLK), lambda i: (0, i)),
            pl.BlockSpec((_IN_DIM, _LATENT), lambda i: (0, 0)),
            pl.BlockSpec((1, _LATENT), lambda i: (0, 0)),
            pl.BlockSpec((_LATENT, _OUT_DIM), lambda i: (0, 0)),
            pl.BlockSpec((1, _OUT_DIM), lambda i: (0, 0)),
        ],
        out_specs=pl.BlockSpec((_BLK, _OUT_DIM), lambda i: (i, 0)),
        out_shape=jax.ShapeDtypeStruct((_N_NODES, _OUT_DIM), jnp.float32),
    )(node_attr, sums, cnts, global_attr, ng2, W1, b1r, W2, b2r)


def kernel(node_attr, edge_attr, global_attr, edge_index, ng_index, eg_index,
           W1, b1, W2, b2):
    e4 = edge_attr.T.reshape(2, 8, _NCH, 128).transpose(0, 2, 1, 3)
    e4 = e4.reshape(2, _NCH, 1024)
    ei3 = edge_index.T.reshape(_NCH, 128, 2).transpose(0, 2, 1)
    ei3 = ei3.reshape(_NCH, 256)
    zeros_n = jnp.zeros((_N_NODES,), jnp.float32)
    sums, cnts = _sc_scatter_mean_partials(e4, ei3, zeros_n)
    ng2 = ng_index.reshape(1, _N_NODES)
    return _tc_mlp(node_attr, sums, cnts, global_attr, ng2,
                   W1, b1.reshape(1, -1), W2, b2.reshape(1, -1))
